# Initial kernel scaffold; baseline (speedup 1.0000x reference)
import jax
import jax.numpy as jnp
from jax.experimental import pallas as pl


def kernel(X, Z):
    raise NotImplementedError("write your pallas kernel here")


# speedup vs baseline: 12.7498x; 12.7498x over previous
import functools

import jax
import jax.numpy as jnp
from jax import lax
from jax.experimental import pallas as pl
from jax.experimental.pallas import tpu as pltpu
from jax.experimental.pallas import tpu_sc as plsc

N = 4096
DX = 512
DZ = 256
BLK = 256
K = 5
EPSF = 1e-07
FLOOR = 1e-24

_NC = 2
_NS = 16
_NW = _NC * _NS
_B = N * (K - 1)
_BPW = _B // _NW
_CHUNK = 128


def _knn_body(xblk_ref, x_ref, lid_ref, idx_ref, b2_scr):
    i = pl.program_id(0)

    @pl.when(i == 0)
    def _():
        xx = x_ref[...] * x_ref[...]
        ones = jnp.ones((1, DX), jnp.float32)
        b2_scr[...] = lax.dot_general(
            ones, xx, (((1,), (1,)), ((), ())),
            preferred_element_type=jnp.float32,
            precision=lax.Precision.HIGHEST)

    xb = xblk_ref[...]
    a2 = jnp.sum(xb * xb, axis=1, keepdims=True)
    dot = lax.dot_general(
        xb, x_ref[...], (((1,), (1,)), ((), ())),
        preferred_element_type=jnp.float32,
        precision=lax.Precision.HIGHEST)
    work = b2_scr[...] - 2.0 * dot

    cols = lax.broadcasted_iota(jnp.int32, (BLK, N), 1)
    vals, idxs = [], []
    for t in range(K):
        m = jnp.min(work, axis=1, keepdims=True)
        cand = jnp.where(work == m, cols, N)
        j = jnp.min(cand, axis=1, keepdims=True)
        vals.append(m)
        idxs.append(j)
        if t < K - 1:
            work = jnp.where(cols == j, jnp.inf, work)

    logs = []
    for t in range(1, K):
        v = jnp.sqrt(jnp.maximum(vals[t] + a2, FLOOR)) + EPSF
        logs.append(jnp.log10(v))
    l_last = logs[-1]
    lid_ref[...] = (1.0 / K) * (
        ((logs[0] - l_last) + (logs[1] - l_last)) + (logs[2] - l_last))
    idx_ref[...] = jnp.concatenate(idxs[1:], axis=1)


_knn_call = pl.pallas_call(
    _knn_body,
    grid=(N // BLK,),
    in_specs=[
        pl.BlockSpec((BLK, DX), lambda i: (i, 0)),
        pl.BlockSpec((N, DX), lambda i: (0, 0)),
    ],
    out_specs=[
        pl.BlockSpec((BLK, 1), lambda i: (i, 0)),
        pl.BlockSpec((BLK, 4), lambda i: (i, 0)),
    ],
    out_shape=[
        jax.ShapeDtypeStruct((N, 1), jnp.float32),
        jax.ShapeDtypeStruct((N, 4), jnp.int32),
    ],
    scratch_shapes=[pltpu.VMEM((1, N), jnp.float32)],
)


@functools.cache
def _sc_gather_call():
    @functools.partial(
        pl.kernel,
        mesh=plsc.VectorSubcoreMesh(core_axis_name="c", subcore_axis_name="s"),
        out_type=jax.ShapeDtypeStruct((_B, DZ), jnp.float32),
        scratch_types=[
            pltpu.VMEM((_CHUNK,), jnp.int32),
            pltpu.VMEM((_CHUNK, DZ), jnp.float32),
            pltpu.SemaphoreType.DMA,
        ],
    )
    def _sc_gather(idx_hbm, z_hbm, out_hbm, idx_v, rows_v, sem):
        wid = lax.axis_index("s") * _NC + lax.axis_index("c")
        base = wid * _BPW
        for c in range(_BPW // _CHUNK):
            off = base + c * _CHUNK
            pltpu.sync_copy(idx_hbm.at[pl.ds(off, _CHUNK)], idx_v)
            pltpu.async_copy(z_hbm.at[idx_v], rows_v, sem).wait()
            pltpu.sync_copy(rows_v, out_hbm.at[pl.ds(off, _CHUNK)])

    return _sc_gather


def _lidz_body(z_ref, g_ref, lidx_ref, out_ref):
    z = z_ref[...]
    z2i = jnp.sum(z * z, axis=1, keepdims=True)
    logs = []
    for t in range(K - 1):
        gt = g_ref[:, t * DZ:(t + 1) * DZ]
        dt = jnp.sum(z * gt, axis=1, keepdims=True)
        z2t = jnp.sum(gt * gt, axis=1, keepdims=True)
        e = jnp.sqrt(jnp.maximum(z2i + z2t - 2.0 * dt, FLOOR)) + EPSF
        logs.append(jnp.log10(e))
    l_last = logs[-1]
    lidz = (1.0 / K) * (
        ((logs[0] - l_last) + (logs[1] - l_last)) + (logs[2] - l_last))
    d = lidx_ref[...] - lidz
    out_ref[...] = jnp.sum(d * d, axis=0, keepdims=True) * (1.0 / N)


_lidz_call = pl.pallas_call(
    _lidz_body,
    out_shape=jax.ShapeDtypeStruct((1, 1), jnp.float32),
)


def kernel(X, Z):
    lid_x, idx4 = _knn_call(X, X)
    ind_flat = idx4.reshape(-1)
    g = _sc_gather_call()(ind_flat, Z)
    g4 = g.reshape(N, (K - 1) * DZ)
    out = _lidz_call(Z, g4, lid_x)
    return out[0, 0]


# speedup vs baseline: 19.0366x; 1.4931x over previous
import functools

import jax
import jax.numpy as jnp
from jax import lax
from jax.experimental import pallas as pl
from jax.experimental.pallas import tpu as pltpu
from jax.experimental.pallas import tpu_sc as plsc

N = 4096
DX = 512
DZ = 256
BLK = 256
K = 5
EPSF = 1e-07
FLOOR = 1e-24

_NC = 2
_NS = 16
_NW = _NC * _NS
_B = N * (K - 1)
_BPW = _B // _NW
_CHUNK = 128


def _knn_body(xblk_ref, x_ref, lid_ref, idx_ref, b2_scr, bhl_scr):
    i = pl.program_id(0)

    @pl.when(i == 0)
    def _():
        xf = x_ref[...]
        xx = xf * xf
        ones = jnp.ones((1, DX), jnp.float32)
        b2_scr[...] = lax.dot_general(
            ones, xx, (((1,), (1,)), ((), ())),
            preferred_element_type=jnp.float32,
            precision=lax.Precision.HIGHEST)
        bh = xf.astype(jnp.bfloat16)
        bl = (xf - bh.astype(jnp.float32)).astype(jnp.bfloat16)
        bhl_scr[...] = jnp.concatenate([bh, bl, bh], axis=1)

    xb = xblk_ref[...]
    a2 = jnp.sum(xb * xb, axis=1, keepdims=True)
    xbm2 = -2.0 * xb
    ah = xbm2.astype(jnp.bfloat16)
    al = (xbm2 - ah.astype(jnp.float32)).astype(jnp.bfloat16)
    ahl = jnp.concatenate([ah, ah, al], axis=1)
    dotm2 = lax.dot_general(
        ahl, bhl_scr[...], (((1,), (1,)), ((), ())),
        preferred_element_type=jnp.float32)
    work = b2_scr[...] + dotm2

    inf = jnp.full((BLK, 128), jnp.inf, jnp.float32)
    m1, m2, m3 = inf, inf, inf
    zc = jnp.zeros((BLK, 128), jnp.int32)
    c1, c2, c3 = zc, zc, zc
    for t in range(N // 128):
        x = work[:, t * 128:(t + 1) * 128]
        ct = jnp.full((BLK, 128), t, jnp.int32)
        t1 = x < m1
        t2 = x < m2
        t3 = x < m3
        m3 = jnp.where(t3, jnp.where(t2, m2, x), m3)
        c3 = jnp.where(t3, jnp.where(t2, c2, ct), c3)
        m2 = jnp.where(t2, jnp.where(t1, m1, x), m2)
        c2 = jnp.where(t2, jnp.where(t1, c1, ct), c2)
        m1 = jnp.where(t1, x, m1)
        c1 = jnp.where(t1, ct, c1)

    lane = lax.broadcasted_iota(jnp.int32, (BLK, 128), 1)
    candv = jnp.concatenate([m1, m2, m3], axis=1)
    candi = jnp.concatenate(
        [c1 * 128 + lane, c2 * 128 + lane, c3 * 128 + lane], axis=1)
    vals, idxs = [], []
    for t in range(K):
        m = jnp.min(candv, axis=1, keepdims=True)
        j = jnp.min(jnp.where(candv == m, candi, N),
                    axis=1, keepdims=True)
        vals.append(m)
        idxs.append(j)
        if t < K - 1:
            candv = jnp.where(candi == j, jnp.inf, candv)

    logs = []
    for t in range(1, K):
        v = jnp.sqrt(jnp.maximum(vals[t] + a2, FLOOR)) + EPSF
        logs.append(jnp.log10(v))
    l_last = logs[-1]
    lid_ref[...] = (1.0 / K) * (
        ((logs[0] - l_last) + (logs[1] - l_last)) + (logs[2] - l_last))
    idx_ref[...] = jnp.concatenate(idxs[1:], axis=1)


_knn_call = pl.pallas_call(
    _knn_body,
    grid=(N // BLK,),
    in_specs=[
        pl.BlockSpec((BLK, DX), lambda i: (i, 0)),
        pl.BlockSpec((N, DX), lambda i: (0, 0)),
    ],
    out_specs=[
        pl.BlockSpec((BLK, 1), lambda i: (i, 0)),
        pl.BlockSpec((BLK, 4), lambda i: (i, 0)),
    ],
    out_shape=[
        jax.ShapeDtypeStruct((N, 1), jnp.float32),
        jax.ShapeDtypeStruct((N, 4), jnp.int32),
    ],
    scratch_shapes=[
        pltpu.VMEM((1, N), jnp.float32),
        pltpu.VMEM((N, 3 * DX), jnp.bfloat16),
    ],
)


@functools.cache
def _sc_gather_call():
    @functools.partial(
        pl.kernel,
        mesh=plsc.VectorSubcoreMesh(core_axis_name="c", subcore_axis_name="s"),
        out_type=jax.ShapeDtypeStruct((_B, DZ), jnp.float32),
        scratch_types=[
            pltpu.VMEM((_CHUNK,), jnp.int32),
            pltpu.VMEM((_CHUNK, DZ), jnp.float32),
            pltpu.SemaphoreType.DMA,
        ],
    )
    def _sc_gather(idx_hbm, z_hbm, out_hbm, idx_v, rows_v, sem):
        wid = lax.axis_index("s") * _NC + lax.axis_index("c")
        base = wid * _BPW
        for c in range(_BPW // _CHUNK):
            off = base + c * _CHUNK
            pltpu.sync_copy(idx_hbm.at[pl.ds(off, _CHUNK)], idx_v)
            pltpu.async_copy(z_hbm.at[idx_v], rows_v, sem).wait()
            pltpu.sync_copy(rows_v, out_hbm.at[pl.ds(off, _CHUNK)])

    return _sc_gather


def _lidz_body(z_ref, g_ref, lidx_ref, out_ref):
    z = z_ref[...]
    z2i = jnp.sum(z * z, axis=1, keepdims=True)
    logs = []
    for t in range(K - 1):
        gt = g_ref[:, t * DZ:(t + 1) * DZ]
        dt = jnp.sum(z * gt, axis=1, keepdims=True)
        z2t = jnp.sum(gt * gt, axis=1, keepdims=True)
        e = jnp.sqrt(jnp.maximum(z2i + z2t - 2.0 * dt, FLOOR)) + EPSF
        logs.append(jnp.log10(e))
    l_last = logs[-1]
    lidz = (1.0 / K) * (
        ((logs[0] - l_last) + (logs[1] - l_last)) + (logs[2] - l_last))
    d = lidx_ref[...] - lidz
    out_ref[...] = jnp.sum(d * d, axis=0, keepdims=True) * (1.0 / N)


_lidz_call = pl.pallas_call(
    _lidz_body,
    out_shape=jax.ShapeDtypeStruct((1, 1), jnp.float32),
)


def kernel(X, Z):
    lid_x, idx4 = _knn_call(X, X)
    ind_flat = idx4.reshape(-1)
    g = _sc_gather_call()(ind_flat, Z)
    g4 = g.reshape(N, (K - 1) * DZ)
    out = _lidz_call(Z, g4, lid_x)
    return out[0, 0]


# speedup vs baseline: 20.9189x; 1.0989x over previous
import functools

import jax
import jax.numpy as jnp
from jax import lax
from jax.experimental import pallas as pl
from jax.experimental.pallas import tpu as pltpu
from jax.experimental.pallas import tpu_sc as plsc

N = 4096
DX = 512
DZ = 256
BLK = 1024
K = 5
EPSF = 1e-07
FLOOR = 1e-24

_NC = 2
_NS = 16
_NW = _NC * _NS
_B = N * (K - 1)
_BPW = _B // _NW
_CHUNK = 128


def _knn_body(xblk_ref, x_ref, lid_ref, idx_ref, b2_scr, bhl_scr):
    i = pl.program_id(0)

    @pl.when(i == 0)
    def _():
        xf = x_ref[...]
        xx = xf * xf
        ones = jnp.ones((1, DX), jnp.float32)
        b2_scr[...] = lax.dot_general(
            ones, xx, (((1,), (1,)), ((), ())),
            preferred_element_type=jnp.float32,
            precision=lax.Precision.HIGHEST)
        bh = xf.astype(jnp.bfloat16)
        bl = (xf - bh.astype(jnp.float32)).astype(jnp.bfloat16)
        bhl_scr[...] = jnp.concatenate([bh, bl, bh], axis=1)

    xb = xblk_ref[...]
    a2 = jnp.sum(xb * xb, axis=1, keepdims=True)
    xbm2 = -2.0 * xb
    ah = xbm2.astype(jnp.bfloat16)
    al = (xbm2 - ah.astype(jnp.float32)).astype(jnp.bfloat16)
    ahl = jnp.concatenate([ah, ah, al], axis=1)
    dotm2 = lax.dot_general(
        ahl, bhl_scr[...], (((1,), (1,)), ((), ())),
        preferred_element_type=jnp.float32)
    work = b2_scr[...] + dotm2

    inf = jnp.full((BLK, 128), jnp.inf, jnp.float32)
    m1, m2, m3 = inf, inf, inf
    zc = jnp.zeros((BLK, 128), jnp.int32)
    c1, c2, c3 = zc, zc, zc
    for t in range(N // 128):
        x = work[:, t * 128:(t + 1) * 128]
        ct = jnp.full((BLK, 128), t, jnp.int32)
        t1 = x < m1
        t2 = x < m2
        t3 = x < m3
        m3 = jnp.where(t3, jnp.where(t2, m2, x), m3)
        c3 = jnp.where(t3, jnp.where(t2, c2, ct), c3)
        m2 = jnp.where(t2, jnp.where(t1, m1, x), m2)
        c2 = jnp.where(t2, jnp.where(t1, c1, ct), c2)
        m1 = jnp.where(t1, x, m1)
        c1 = jnp.where(t1, ct, c1)

    lane = lax.broadcasted_iota(jnp.int32, (BLK, 128), 1)
    candv = jnp.concatenate([m1, m2, m3], axis=1)
    candi = jnp.concatenate(
        [c1 * 128 + lane, c2 * 128 + lane, c3 * 128 + lane], axis=1)
    vals, idxs = [], []
    for t in range(K):
        m = jnp.min(candv, axis=1, keepdims=True)
        j = jnp.min(jnp.where(candv == m, candi, N),
                    axis=1, keepdims=True)
        vals.append(m)
        idxs.append(j)
        if t < K - 1:
            candv = jnp.where(candi == j, jnp.inf, candv)

    logs = []
    for t in range(1, K):
        v = jnp.sqrt(jnp.maximum(vals[t] + a2, FLOOR)) + EPSF
        logs.append(jnp.log10(v))
    l_last = logs[-1]
    lid_ref[...] = (1.0 / K) * (
        ((logs[0] - l_last) + (logs[1] - l_last)) + (logs[2] - l_last))
    idx_ref[...] = jnp.concatenate(idxs[1:], axis=1)


_knn_call = pl.pallas_call(
    _knn_body,
    grid=(N // BLK,),
    in_specs=[
        pl.BlockSpec((BLK, DX), lambda i: (i, 0)),
        pl.BlockSpec((N, DX), lambda i: (0, 0)),
    ],
    out_specs=[
        pl.BlockSpec((BLK, 1), lambda i: (i, 0)),
        pl.BlockSpec((BLK, 4), lambda i: (i, 0)),
    ],
    out_shape=[
        jax.ShapeDtypeStruct((N, 1), jnp.float32),
        jax.ShapeDtypeStruct((N, 4), jnp.int32),
    ],
    scratch_shapes=[
        pltpu.VMEM((1, N), jnp.float32),
        pltpu.VMEM((N, 3 * DX), jnp.bfloat16),
    ],
)


@functools.cache
def _sc_gather_call():
    @functools.partial(
        pl.kernel,
        mesh=plsc.VectorSubcoreMesh(core_axis_name="c", subcore_axis_name="s"),
        out_type=jax.ShapeDtypeStruct((_B, DZ), jnp.float32),
        scratch_types=[
            pltpu.VMEM((_CHUNK,), jnp.int32),
            pltpu.VMEM((_CHUNK,), jnp.int32),
            pltpu.VMEM((_CHUNK, DZ), jnp.float32),
            pltpu.VMEM((_CHUNK, DZ), jnp.float32),
            pltpu.SemaphoreType.DMA,
            pltpu.SemaphoreType.DMA,
            pltpu.SemaphoreType.DMA,
            pltpu.SemaphoreType.DMA,
        ],
    )
    def _sc_gather(idx_hbm, z_hbm, out_hbm, idx_a, idx_b, rows_a, rows_b,
                   gsem_a, gsem_b, osem_a, osem_b):
        wid = lax.axis_index("s") * _NC + lax.axis_index("c")
        base = wid * _BPW
        nch = _BPW // _CHUNK
        idx_v = [idx_a, idx_b]
        rows_v = [rows_a, rows_b]
        gsem = [gsem_a, gsem_b]
        osem = [osem_a, osem_b]
        pltpu.sync_copy(idx_hbm.at[pl.ds(base, _CHUNK)], idx_a)
        gcp = [pltpu.async_copy(z_hbm.at[idx_a], rows_a, gsem_a), None]
        ocp = [None, None]
        for c in range(nch):
            cur = c % 2
            nxt = (c + 1) % 2
            if c + 1 < nch:
                if ocp[nxt] is not None:
                    ocp[nxt].wait()
                    ocp[nxt] = None
                off_n = base + (c + 1) * _CHUNK
                pltpu.sync_copy(idx_hbm.at[pl.ds(off_n, _CHUNK)], idx_v[nxt])
                gcp[nxt] = pltpu.async_copy(
                    z_hbm.at[idx_v[nxt]], rows_v[nxt], gsem[nxt])
            gcp[cur].wait()
            off = base + c * _CHUNK
            ocp[cur] = pltpu.async_copy(
                rows_v[cur], out_hbm.at[pl.ds(off, _CHUNK)], osem[cur])
        for c in range(2):
            if ocp[c] is not None:
                ocp[c].wait()

    return _sc_gather


def _lidz_body(z_ref, g_ref, lidx_ref, out_ref):
    z = z_ref[...]
    z2i = jnp.sum(z * z, axis=1, keepdims=True)
    logs = []
    for t in range(K - 1):
        gt = g_ref[:, t * DZ:(t + 1) * DZ]
        dt = jnp.sum(z * gt, axis=1, keepdims=True)
        z2t = jnp.sum(gt * gt, axis=1, keepdims=True)
        e = jnp.sqrt(jnp.maximum(z2i + z2t - 2.0 * dt, FLOOR)) + EPSF
        logs.append(jnp.log10(e))
    l_last = logs[-1]
    lidz = (1.0 / K) * (
        ((logs[0] - l_last) + (logs[1] - l_last)) + (logs[2] - l_last))
    d = lidx_ref[...] - lidz
    out_ref[...] = jnp.sum(d * d, axis=0, keepdims=True) * (1.0 / N)


_lidz_call = pl.pallas_call(
    _lidz_body,
    out_shape=jax.ShapeDtypeStruct((1, 1), jnp.float32),
)


def kernel(X, Z):
    lid_x, idx4 = _knn_call(X, X)
    ind_flat = idx4.reshape(-1)
    g = _sc_gather_call()(ind_flat, Z)
    g4 = g.reshape(N, (K - 1) * DZ)
    out = _lidz_call(Z, g4, lid_x)
    return out[0, 0]


# speedup vs baseline: 23.6133x; 1.1288x over previous
import functools

import jax
import jax.numpy as jnp
from jax import lax
from jax.experimental import pallas as pl
from jax.experimental.pallas import tpu as pltpu
from jax.experimental.pallas import tpu_sc as plsc

N = 4096
DX = 512
DZ = 256
BLK = 1024
K = 5
EPSF = 1e-07
FLOOR = 1e-24

_NC = 2
_NS = 16
_NW = _NC * _NS
_B = N * (K - 1)
_BPW = _B // _NW
_CHUNK = 128


def _knn_body(xblk_ref, x_ref, lid_ref, idx_ref, b2_scr, bhl_scr):
    i = pl.program_id(0)

    @pl.when(i == 0)
    def _():
        xf = x_ref[...]
        xx = xf * xf
        ones = jnp.ones((1, DX), jnp.float32)
        b2_scr[...] = lax.dot_general(
            ones, xx, (((1,), (1,)), ((), ())),
            preferred_element_type=jnp.float32,
            precision=lax.Precision.HIGHEST)
        bh = xf.astype(jnp.bfloat16)
        bl = (xf - bh.astype(jnp.float32)).astype(jnp.bfloat16)
        bhl_scr[...] = jnp.concatenate([bh, bl, bh], axis=1)

    xb = xblk_ref[...]
    a2 = jnp.sum(xb * xb, axis=1, keepdims=True)
    xbm2 = -2.0 * xb
    ah = xbm2.astype(jnp.bfloat16)
    al = (xbm2 - ah.astype(jnp.float32)).astype(jnp.bfloat16)
    ahl = jnp.concatenate([ah, ah, al], axis=1)
    dotm2 = lax.dot_general(
        ahl, bhl_scr[...], (((1,), (1,)), ((), ())),
        preferred_element_type=jnp.float32)
    work = b2_scr[...] + dotm2

    inf = jnp.full((BLK, 128), jnp.inf, jnp.float32)
    m1, m2, m3 = inf, inf, inf
    zc = jnp.zeros((BLK, 128), jnp.int32)
    c1, c2, c3 = zc, zc, zc
    for t in range(N // 128):
        x = work[:, t * 128:(t + 1) * 128]
        ct = jnp.full((BLK, 128), t, jnp.int32)
        t1 = x < m1
        t2 = x < m2
        t3 = x < m3
        m3 = jnp.where(t3, jnp.where(t2, m2, x), m3)
        c3 = jnp.where(t3, jnp.where(t2, c2, ct), c3)
        m2 = jnp.where(t2, jnp.where(t1, m1, x), m2)
        c2 = jnp.where(t2, jnp.where(t1, c1, ct), c2)
        m1 = jnp.where(t1, x, m1)
        c1 = jnp.where(t1, ct, c1)

    lane = lax.broadcasted_iota(jnp.int32, (BLK, 128), 1)
    candv = jnp.concatenate([m1, m2, m3], axis=1)
    candi = jnp.concatenate(
        [c1 * 128 + lane, c2 * 128 + lane, c3 * 128 + lane], axis=1)
    vals, idxs = [], []
    for t in range(K):
        m = jnp.min(candv, axis=1, keepdims=True)
        j = jnp.min(jnp.where(candv == m, candi, N),
                    axis=1, keepdims=True)
        vals.append(m)
        idxs.append(j)
        if t < K - 1:
            candv = jnp.where(candi == j, jnp.inf, candv)

    logs = []
    for t in range(1, K):
        v = jnp.sqrt(jnp.maximum(vals[t] + a2, FLOOR)) + EPSF
        logs.append(jnp.log10(v))
    l_last = logs[-1]
    lid_ref[...] = (1.0 / K) * (
        ((logs[0] - l_last) + (logs[1] - l_last)) + (logs[2] - l_last))
    idx_ref[...] = jnp.concatenate(idxs[1:], axis=1)


_knn_call = pl.pallas_call(
    _knn_body,
    grid=(N // BLK,),
    in_specs=[
        pl.BlockSpec((BLK, DX), lambda i: (i, 0)),
        pl.BlockSpec((N, DX), lambda i: (0, 0)),
    ],
    out_specs=[
        pl.BlockSpec((BLK, 1), lambda i: (i, 0)),
        pl.BlockSpec((BLK, 4), lambda i: (i, 0)),
    ],
    out_shape=[
        jax.ShapeDtypeStruct((N, 1), jnp.float32),
        jax.ShapeDtypeStruct((N, 4), jnp.int32),
    ],
    scratch_shapes=[
        pltpu.VMEM((1, N), jnp.float32),
        pltpu.VMEM((N, 3 * DX), jnp.bfloat16),
    ],
)


@functools.cache
def _sc_gather_call():
    @functools.partial(
        pl.kernel,
        mesh=plsc.VectorSubcoreMesh(core_axis_name="c", subcore_axis_name="s"),
        out_type=jax.ShapeDtypeStruct((_B, DZ), jnp.float32),
        scratch_types=[
            pltpu.VMEM((_CHUNK,), jnp.int32),
            pltpu.VMEM((_CHUNK,), jnp.int32),
            pltpu.VMEM((_CHUNK, DZ), jnp.float32),
            pltpu.VMEM((_CHUNK, DZ), jnp.float32),
            pltpu.SemaphoreType.DMA,
            pltpu.SemaphoreType.DMA,
            pltpu.SemaphoreType.DMA,
            pltpu.SemaphoreType.DMA,
        ],
    )
    def _sc_gather(idx_hbm, z_hbm, out_hbm, idx_a, idx_b, rows_a, rows_b,
                   gsem_a, gsem_b, osem_a, osem_b):
        wid = lax.axis_index("s") * _NC + lax.axis_index("c")
        base = wid * _BPW
        nch = _BPW // _CHUNK
        idx_v = [idx_a, idx_b]
        rows_v = [rows_a, rows_b]
        gsem = [gsem_a, gsem_b]
        osem = [osem_a, osem_b]
        pltpu.sync_copy(idx_hbm.at[pl.ds(base, _CHUNK)], idx_a)
        gcp = [pltpu.async_copy(z_hbm.at[idx_a], rows_a, gsem_a), None]
        ocp = [None, None]
        for c in range(nch):
            cur = c % 2
            nxt = (c + 1) % 2
            if c + 1 < nch:
                if ocp[nxt] is not None:
                    ocp[nxt].wait()
                    ocp[nxt] = None
                off_n = base + (c + 1) * _CHUNK
                pltpu.sync_copy(idx_hbm.at[pl.ds(off_n, _CHUNK)], idx_v[nxt])
                gcp[nxt] = pltpu.async_copy(
                    z_hbm.at[idx_v[nxt]], rows_v[nxt], gsem[nxt])
            gcp[cur].wait()
            off = base + c * _CHUNK
            ocp[cur] = pltpu.async_copy(
                rows_v[cur], out_hbm.at[pl.ds(off, _CHUNK)], osem[cur])
        for c in range(2):
            if ocp[c] is not None:
                ocp[c].wait()

    return _sc_gather


def _lidz_body(z_ref, g_ref, lidx_ref, out_ref):
    z = z_ref[...]
    z2i = jnp.sum(z * z, axis=1, keepdims=True)
    logs = []
    for t in range(K - 1):
        gt = g_ref[t * N:(t + 1) * N, :]
        dt = jnp.sum(z * gt, axis=1, keepdims=True)
        z2t = jnp.sum(gt * gt, axis=1, keepdims=True)
        e = jnp.sqrt(jnp.maximum(z2i + z2t - 2.0 * dt, FLOOR)) + EPSF
        logs.append(jnp.log10(e))
    l_last = logs[-1]
    lidz = (1.0 / K) * (
        ((logs[0] - l_last) + (logs[1] - l_last)) + (logs[2] - l_last))
    d = lidx_ref[...] - lidz
    out_ref[...] = jnp.sum(d * d, axis=0, keepdims=True) * (1.0 / N)


_lidz_call = pl.pallas_call(
    _lidz_body,
    out_shape=jax.ShapeDtypeStruct((1, 1), jnp.float32),
)


def kernel(X, Z):
    lid_x, idx4 = _knn_call(X, X)
    ind_t = idx4.T.reshape(-1)
    g = _sc_gather_call()(ind_t, Z)
    out = _lidz_call(Z, g, lid_x)
    return out[0, 0]


# speedup vs baseline: 23.7426x; 1.0055x over previous
import functools

import jax
import jax.numpy as jnp
from jax import lax
from jax.experimental import pallas as pl
from jax.experimental.pallas import tpu as pltpu
from jax.experimental.pallas import tpu_sc as plsc

N = 4096
DX = 512
DZ = 256
BLK = 1024
K = 5
EPSF = 1e-07
FLOOR = 1e-24

_NC = 2
_NS = 16
_NW = _NC * _NS
_B = N * (K - 1)
_BPW = _B // _NW
_CHUNK = 128


def _knn_body(xblk_ref, x_ref, lid_ref, idx_ref, b2_scr, bhl_scr):
    i = pl.program_id(0)

    @pl.when(i == 0)
    def _():
        xf = x_ref[...]
        xx = xf * xf
        ones = jnp.ones((1, DX), jnp.float32)
        b2_scr[...] = lax.dot_general(
            ones, xx, (((1,), (1,)), ((), ())),
            preferred_element_type=jnp.float32,
            precision=lax.Precision.HIGHEST)
        bh = xf.astype(jnp.bfloat16)
        bl = (xf - bh.astype(jnp.float32)).astype(jnp.bfloat16)
        bhl_scr[...] = jnp.concatenate([bh, bl, bh], axis=1)

    xb = xblk_ref[...]
    a2 = jnp.sum(xb * xb, axis=1, keepdims=True)
    xbm2 = -2.0 * xb
    ah = xbm2.astype(jnp.bfloat16)
    al = (xbm2 - ah.astype(jnp.float32)).astype(jnp.bfloat16)
    ahl = jnp.concatenate([ah, ah, al], axis=1)
    dotm2 = lax.dot_general(
        ahl, bhl_scr[...], (((1,), (1,)), ((), ())),
        preferred_element_type=jnp.float32)
    work = b2_scr[...] + dotm2

    inf = jnp.full((BLK, 128), jnp.inf, jnp.float32)
    m1, m2, m3 = inf, inf, inf
    zc = jnp.zeros((BLK, 128), jnp.int32)
    c1, c2, c3 = zc, zc, zc
    for t in range(N // 128):
        x = work[:, t * 128:(t + 1) * 128]
        ct = jnp.full((BLK, 128), t, jnp.int32)
        t1 = x < m1
        t2 = x < m2
        t3 = x < m3
        m3 = jnp.where(t3, jnp.where(t2, m2, x), m3)
        c3 = jnp.where(t3, jnp.where(t2, c2, ct), c3)
        m2 = jnp.where(t2, jnp.where(t1, m1, x), m2)
        c2 = jnp.where(t2, jnp.where(t1, c1, ct), c2)
        m1 = jnp.where(t1, x, m1)
        c1 = jnp.where(t1, ct, c1)

    lane = lax.broadcasted_iota(jnp.int32, (BLK, 128), 1)
    candv = jnp.concatenate([m1, m2, m3], axis=1)
    candi = jnp.concatenate(
        [c1 * 128 + lane, c2 * 128 + lane, c3 * 128 + lane], axis=1)
    vals, idxs = [], []
    for t in range(K):
        m = jnp.min(candv, axis=1, keepdims=True)
        j = jnp.min(jnp.where(candv == m, candi, N),
                    axis=1, keepdims=True)
        vals.append(m)
        idxs.append(j)
        if t < K - 1:
            candv = jnp.where(candi == j, jnp.inf, candv)

    logs = []
    for t in range(1, K):
        v = jnp.sqrt(jnp.maximum(vals[t] + a2, FLOOR)) + EPSF
        logs.append(jnp.log10(v))
    l_last = logs[-1]
    lid_ref[...] = (1.0 / K) * (
        ((logs[0] - l_last) + (logs[1] - l_last)) + (logs[2] - l_last))
    idx_ref[...] = jnp.concatenate(idxs[1:], axis=1)


_knn_call = pl.pallas_call(
    _knn_body,
    grid=(N // BLK,),
    in_specs=[
        pl.BlockSpec((BLK, DX), lambda i: (i, 0)),
        pl.BlockSpec((N, DX), lambda i: (0, 0)),
    ],
    out_specs=[
        pl.BlockSpec((BLK, 1), lambda i: (i, 0)),
        pl.BlockSpec((BLK, 4), lambda i: (i, 0)),
    ],
    out_shape=[
        jax.ShapeDtypeStruct((N, 1), jnp.float32),
        jax.ShapeDtypeStruct((N, 4), jnp.int32),
    ],
    scratch_shapes=[
        pltpu.VMEM((1, N), jnp.float32),
        pltpu.VMEM((N, 3 * DX), jnp.bfloat16),
    ],
)


@functools.cache
def _sc_gather_call():
    @functools.partial(
        pl.kernel,
        mesh=plsc.VectorSubcoreMesh(core_axis_name="c", subcore_axis_name="s"),
        out_type=jax.ShapeDtypeStruct((_B, DZ), jnp.float32),
        scratch_types=[
            pltpu.VMEM((_BPW,), jnp.int32),
            pltpu.VMEM((_CHUNK, DZ), jnp.float32),
            pltpu.VMEM((_CHUNK, DZ), jnp.float32),
            pltpu.SemaphoreType.DMA,
            pltpu.SemaphoreType.DMA,
            pltpu.SemaphoreType.DMA,
            pltpu.SemaphoreType.DMA,
        ],
    )
    def _sc_gather(idx_hbm, z_hbm, out_hbm, idx_all, rows_a, rows_b,
                   gsem_a, gsem_b, osem_a, osem_b):
        wid = lax.axis_index("s") * _NC + lax.axis_index("c")
        base = wid * _BPW
        nch = _BPW // _CHUNK
        rows_v = [rows_a, rows_b]
        gsem = [gsem_a, gsem_b]
        osem = [osem_a, osem_b]
        pltpu.sync_copy(idx_hbm.at[pl.ds(base, _BPW)], idx_all)
        gcp = [pltpu.async_copy(
            z_hbm.at[idx_all.at[pl.ds(0, _CHUNK)]], rows_a, gsem_a), None]
        ocp = [None, None]
        for c in range(nch):
            cur = c % 2
            nxt = (c + 1) % 2
            if c + 1 < nch:
                if ocp[nxt] is not None:
                    ocp[nxt].wait()
                    ocp[nxt] = None
                gcp[nxt] = pltpu.async_copy(
                    z_hbm.at[idx_all.at[pl.ds((c + 1) * _CHUNK, _CHUNK)]],
                    rows_v[nxt], gsem[nxt])
            gcp[cur].wait()
            off = base + c * _CHUNK
            ocp[cur] = pltpu.async_copy(
                rows_v[cur], out_hbm.at[pl.ds(off, _CHUNK)], osem[cur])
        for c in range(2):
            if ocp[c] is not None:
                ocp[c].wait()

    return _sc_gather


def _lidz_body(z_ref, g_ref, lidx_ref, out_ref):
    z = z_ref[...]
    z2i = jnp.sum(z * z, axis=1, keepdims=True)
    logs = []
    for t in range(K - 1):
        gt = g_ref[t * N:(t + 1) * N, :]
        dt = jnp.sum(z * gt, axis=1, keepdims=True)
        z2t = jnp.sum(gt * gt, axis=1, keepdims=True)
        e = jnp.sqrt(jnp.maximum(z2i + z2t - 2.0 * dt, FLOOR)) + EPSF
        logs.append(jnp.log10(e))
    l_last = logs[-1]
    lidz = (1.0 / K) * (
        ((logs[0] - l_last) + (logs[1] - l_last)) + (logs[2] - l_last))
    d = lidx_ref[...] - lidz
    out_ref[...] = jnp.sum(d * d, axis=0, keepdims=True) * (1.0 / N)


_lidz_call = pl.pallas_call(
    _lidz_body,
    out_shape=jax.ShapeDtypeStruct((1, 1), jnp.float32),
)


def kernel(X, Z):
    lid_x, idx4 = _knn_call(X, X)
    ind_t = idx4.T.reshape(-1)
    g = _sc_gather_call()(ind_t, Z)
    out = _lidz_call(Z, g, lid_x)
    return out[0, 0]


# speedup vs baseline: 27.3317x; 1.1512x over previous
import functools

import jax
import jax.numpy as jnp
from jax import lax
from jax.experimental import pallas as pl
from jax.experimental.pallas import tpu as pltpu
from jax.experimental.pallas import tpu_sc as plsc

N = 4096
DX = 512
DZ = 256
BLK = 1024
K = 5
EPSF = 1e-07
FLOOR = 1e-24

_NC = 2
_NS = 16
_NW = _NC * _NS
_B = N * (K - 1)
_BPW = _B // _NW
_CHUNK = 128


def _knn_body(xblk_ref, x_ref, lid_ref, idx_ref, b2_scr):
    i = pl.program_id(0)

    @pl.when(i == 0)
    def _():
        xf = x_ref[...]
        xx = xf * xf
        ones = jnp.ones((1, DX), jnp.float32)
        b2_scr[...] = lax.dot_general(
            ones, xx, (((1,), (1,)), ((), ())),
            preferred_element_type=jnp.float32,
            precision=lax.Precision.HIGHEST)

    xb = xblk_ref[...]
    a2 = jnp.sum(xb * xb, axis=1, keepdims=True)
    dotm2 = lax.dot_general(
        -2.0 * xb, x_ref[...], (((1,), (1,)), ((), ())),
        preferred_element_type=jnp.float32)
    work = b2_scr[...] + dotm2

    inf = jnp.full((BLK, 128), jnp.inf, jnp.float32)
    m1, m2, m3 = inf, inf, inf
    zc = jnp.zeros((BLK, 128), jnp.int32)
    c1, c2, c3 = zc, zc, zc
    for t in range(N // 128):
        x = work[:, t * 128:(t + 1) * 128]
        ct = jnp.full((BLK, 128), t, jnp.int32)
        t1 = x < m1
        t2 = x < m2
        t3 = x < m3
        m3 = jnp.where(t3, jnp.where(t2, m2, x), m3)
        c3 = jnp.where(t3, jnp.where(t2, c2, ct), c3)
        m2 = jnp.where(t2, jnp.where(t1, m1, x), m2)
        c2 = jnp.where(t2, jnp.where(t1, c1, ct), c2)
        m1 = jnp.where(t1, x, m1)
        c1 = jnp.where(t1, ct, c1)

    lane = lax.broadcasted_iota(jnp.int32, (BLK, 128), 1)
    candv = jnp.concatenate([m1, m2, m3], axis=1)
    candi = jnp.concatenate(
        [c1 * 128 + lane, c2 * 128 + lane, c3 * 128 + lane], axis=1)
    vals, idxs = [], []
    for t in range(K):
        m = jnp.min(candv, axis=1, keepdims=True)
        j = jnp.min(jnp.where(candv == m, candi, N),
                    axis=1, keepdims=True)
        vals.append(m)
        idxs.append(j)
        if t < K - 1:
            candv = jnp.where(candi == j, jnp.inf, candv)

    logs = []
    for t in range(1, K):
        v = jnp.sqrt(jnp.maximum(vals[t] + a2, FLOOR)) + EPSF
        logs.append(jnp.log10(v))
    l_last = logs[-1]
    lid_ref[...] = (1.0 / K) * (
        ((logs[0] - l_last) + (logs[1] - l_last)) + (logs[2] - l_last))
    idx_ref[...] = jnp.concatenate(idxs[1:], axis=1)


_knn_call = pl.pallas_call(
    _knn_body,
    grid=(N // BLK,),
    in_specs=[
        pl.BlockSpec((BLK, DX), lambda i: (i, 0)),
        pl.BlockSpec((N, DX), lambda i: (0, 0)),
    ],
    out_specs=[
        pl.BlockSpec((BLK, 1), lambda i: (i, 0)),
        pl.BlockSpec((BLK, 4), lambda i: (i, 0)),
    ],
    out_shape=[
        jax.ShapeDtypeStruct((N, 1), jnp.float32),
        jax.ShapeDtypeStruct((N, 4), jnp.int32),
    ],
    scratch_shapes=[pltpu.VMEM((1, N), jnp.float32)],
)


@functools.cache
def _sc_gather_call():
    @functools.partial(
        pl.kernel,
        mesh=plsc.VectorSubcoreMesh(core_axis_name="c", subcore_axis_name="s"),
        out_type=jax.ShapeDtypeStruct((_B, DZ), jnp.float32),
        scratch_types=[
            pltpu.VMEM((_BPW,), jnp.int32),
            pltpu.VMEM((_CHUNK, DZ), jnp.float32),
            pltpu.VMEM((_CHUNK, DZ), jnp.float32),
            pltpu.SemaphoreType.DMA,
            pltpu.SemaphoreType.DMA,
            pltpu.SemaphoreType.DMA,
            pltpu.SemaphoreType.DMA,
        ],
    )
    def _sc_gather(idx_hbm, z_hbm, out_hbm, idx_all, rows_a, rows_b,
                   gsem_a, gsem_b, osem_a, osem_b):
        wid = lax.axis_index("s") * _NC + lax.axis_index("c")
        base = wid * _BPW
        nch = _BPW // _CHUNK
        rows_v = [rows_a, rows_b]
        gsem = [gsem_a, gsem_b]
        osem = [osem_a, osem_b]
        pltpu.sync_copy(idx_hbm.at[pl.ds(base, _BPW)], idx_all)
        gcp = [pltpu.async_copy(
            z_hbm.at[idx_all.at[pl.ds(0, _CHUNK)]], rows_a, gsem_a), None]
        ocp = [None, None]
        for c in range(nch):
            cur = c % 2
            nxt = (c + 1) % 2
            if c + 1 < nch:
                if ocp[nxt] is not None:
                    ocp[nxt].wait()
                    ocp[nxt] = None
                gcp[nxt] = pltpu.async_copy(
                    z_hbm.at[idx_all.at[pl.ds((c + 1) * _CHUNK, _CHUNK)]],
                    rows_v[nxt], gsem[nxt])
            gcp[cur].wait()
            off = base + c * _CHUNK
            ocp[cur] = pltpu.async_copy(
                rows_v[cur], out_hbm.at[pl.ds(off, _CHUNK)], osem[cur])
        for c in range(2):
            if ocp[c] is not None:
                ocp[c].wait()

    return _sc_gather


def _lidz_body(z_ref, g_ref, lidx_ref, out_ref):
    z = z_ref[...]
    z2i = jnp.sum(z * z, axis=1, keepdims=True)
    logs = []
    for t in range(K - 1):
        gt = g_ref[t * N:(t + 1) * N, :]
        dt = jnp.sum(z * gt, axis=1, keepdims=True)
        z2t = jnp.sum(gt * gt, axis=1, keepdims=True)
        e = jnp.sqrt(jnp.maximum(z2i + z2t - 2.0 * dt, FLOOR)) + EPSF
        logs.append(jnp.log10(e))
    l_last = logs[-1]
    lidz = (1.0 / K) * (
        ((logs[0] - l_last) + (logs[1] - l_last)) + (logs[2] - l_last))
    d = lidx_ref[...] - lidz
    out_ref[...] = jnp.sum(d * d, axis=0, keepdims=True) * (1.0 / N)


_lidz_call = pl.pallas_call(
    _lidz_body,
    out_shape=jax.ShapeDtypeStruct((1, 1), jnp.float32),
)


def kernel(X, Z):
    lid_x, idx4 = _knn_call(X, X)
    ind_t = idx4.T.reshape(-1)
    g = _sc_gather_call()(ind_t, Z)
    out = _lidz_call(Z, g, lid_x)
    return out[0, 0]
